# baseline (speedup 1.0000x reference)
import functools

import jax
import jax.numpy as jnp
from jax import lax
from jax.experimental import pallas as pl
from jax.experimental.pallas import tpu as pltpu
from jax.experimental.pallas import tpu_sc as plsc

N = 10000
F = 128
E = 320000
G = 512
K = 7
NC, NS = 2, 16
NW = NC * NS
CH = 128
EPT = 10240
EPAD = NW * EPT
NROW = 10240
RPT = NROW // NS
DUMP = N
GPT = G // NW
EPT0 = 20480
EPT1 = 0
NCH0 = EPT0 // CH
NCH1 = EPT1 // CH
NB = 2
GI = 16
CW = 128


def _mesh():
    return plsc.VectorSubcoreMesh(
        core_axis_name="c", subcore_axis_name="s", num_cores=NC,
        num_subcores=NS)


def _sc_edge_body(h_hbm, src2_hbm, dst2_hbm, p_hbm, idx_s2, idx_d2, rows,
                  acc, sem0, sem1):
    sems = (sem0, sem1)
    c = lax.axis_index("c")
    s = lax.axis_index("s")
    row0 = s * RPT
    z16 = jnp.zeros((16,), jnp.float32)

    def zrow(i, _):
        for k8 in range(F // 16):
            rows[0, i, pl.ds(k8 * 16, 16)] = z16
        return 0

    lax.fori_loop(0, CH, zrow, 0)
    for b in range(RPT // CH):
        pltpu.sync_copy(rows.at[0], acc.at[pl.ds(row0 + b * CH, CH)])

    plsc.subcore_barrier()
    nch = jnp.where(c == 0, NCH0, NCH1)
    cbase = jnp.where(c == 0, s * NCH0, NS * NCH0 + s * NCH1)

    def group(g, _):
        cb = cbase + g * GI
        pltpu.sync_copy(src2_hbm.at[pl.ds(cb, GI)], idx_s2)
        pltpu.sync_copy(dst2_hbm.at[pl.ds(cb, GI)], idx_d2)
        for b in range(NB):
            pltpu.async_copy(h_hbm.at[idx_s2.at[b]], rows.at[b], sems[b])

        def inner(j2, _):
            for b in range(NB):
                j = j2 * NB + b
                pltpu.make_async_copy(h_hbm.at[idx_s2.at[j]], rows.at[b],
                                      sems[b]).wait()
                pltpu.sync_copy(rows.at[b], acc.at[idx_d2.at[j]], add=True)
                jn = j + NB

                @pl.when(jn < GI)
                def _():
                    pltpu.async_copy(h_hbm.at[idx_s2.at[jn]], rows.at[b],
                                     sems[b])
            return 0

        lax.fori_loop(0, GI // NB, inner, 0)
        return 0

    lax.fori_loop(0, nch // GI, group, 0)
    plsc.subcore_barrier()
    pltpu.sync_copy(acc.at[pl.ds(row0, RPT)], p_hbm.at[c, pl.ds(row0, RPT)])


def _sc_count_body(dst_hbm, cnt_hbm, idx_d, obuf, cacc):
    c = lax.axis_index("c")
    s = lax.axis_index("s")
    wid = c * NS + s
    row0 = s * RPT
    z16 = jnp.zeros((16,), jnp.float32)

    def zo(i, _):
        for k8 in range(CW // 16):
            obuf[i, pl.ds(k8 * 16, 16)] = z16
        return 0

    lax.fori_loop(0, CH, zo, 0)
    for b in range(RPT // CH):
        pltpu.sync_copy(obuf, cacc.at[pl.ds(row0 + b * CH, CH)])

    def so(i, _):
        for k8 in range(CW // 16):
            obuf[i, pl.ds(k8 * 16, 16)] = jnp.ones((16,), jnp.float32)
        return 0

    lax.fori_loop(0, CH, so, 0)
    plsc.subcore_barrier()
    ebase = wid * EPT

    def chunk(i, _):
        eb = ebase + i * CH
        pltpu.sync_copy(dst_hbm.at[pl.ds(eb, CH)], idx_d)
        pltpu.sync_copy(obuf, cacc.at[idx_d], add=True)
        return 0

    lax.fori_loop(0, EPT // CH, chunk, 0)
    plsc.subcore_barrier()
    pltpu.sync_copy(cacc.at[pl.ds(row0, RPT)],
                    cnt_hbm.at[c, pl.ds(row0, RPT)])


def _sc_count(dstp):
    fn = pl.kernel(
        _sc_count_body,
        out_type=jax.ShapeDtypeStruct((NC, NROW, CW), jnp.float32),
        mesh=_mesh(),
        scratch_types=[
            pltpu.VMEM((CH,), jnp.int32),
            pltpu.VMEM((CH, CW), jnp.float32),
            pltpu.VMEM_SHARED((NROW, CW), jnp.float32),
        ],
        compiler_params=pltpu.CompilerParams(needs_layout_passes=False),
    )
    return fn(dstp)


def _sc_edge_agg(h, src2, dst2):
    fn = pl.kernel(
        _sc_edge_body,
        out_type=jax.ShapeDtypeStruct((NC, NROW, F), jnp.float32),
        mesh=_mesh(),
        scratch_types=[
            pltpu.VMEM((GI, CH), jnp.int32),
            pltpu.VMEM((GI, CH), jnp.int32),
            pltpu.VMEM((NB, CH, F), jnp.float32),
            pltpu.VMEM_SHARED((NROW, F), jnp.float32),
            pltpu.SemaphoreType.DMA,
            pltpu.SemaphoreType.DMA,
        ],
        compiler_params=pltpu.CompilerParams(needs_layout_passes=False),
    )
    return fn(h, src2, dst2)


def _tc_sage_body(p_ref, c_ref, x_ref, wl_ref, wr_ref, bl_ref, o_ref):
    p = p_ref[0] + p_ref[1]
    cc = c_ref[0, :, 0:1] + c_ref[1, :, 0:1]
    agg = p / jnp.maximum(cc, 1.0)
    acc = lax.dot_general(agg, wl_ref[...], (((1,), (1,)), ((), ())),
                          preferred_element_type=jnp.float32)
    acc = acc + lax.dot_general(x_ref[...], wr_ref[...],
                                (((1,), (1,)), ((), ())),
                                preferred_element_type=jnp.float32)
    o_ref[...] = jnp.maximum(acc + bl_ref[...], 0.0)


def _tc_sage(P, CNT, x, wl, bl, wr):
    nb = 10
    br = N // nb
    return pl.pallas_call(
        _tc_sage_body,
        grid=(nb,),
        in_specs=[
            pl.BlockSpec((NC, br, F), lambda i: (0, i, 0)),
            pl.BlockSpec((NC, br, CW), lambda i: (0, i, 0)),
            pl.BlockSpec((br, F), lambda i: (i, 0)),
            pl.BlockSpec((F, F), lambda i: (0, 0)),
            pl.BlockSpec((F, F), lambda i: (0, 0)),
            pl.BlockSpec((1, F), lambda i: (0, 0)),
        ],
        out_specs=pl.BlockSpec((br, F), lambda i: (i, 0)),
        out_shape=jax.ShapeDtypeStruct((N, F), jnp.float32),
    )(P, CNT, x, wl, wr, bl.reshape(1, F))


def _tc_counts_body(b_ref, o_ref):
    gids = lax.broadcasted_iota(jnp.int32, (1, G), 1)
    cnt = jnp.zeros((1, G), jnp.float32)
    for i in range(10):
        bi = b_ref[i]
        eq = (bi[0][:, None] == gids).astype(jnp.float32)
        cnt = cnt + jnp.sum(eq, axis=0, keepdims=True)
    rl = lax.broadcasted_iota(jnp.int32, (G, G), 0)
    cl = lax.broadcasted_iota(jnp.int32, (G, G), 1)
    tri = (rl < cl).astype(jnp.float32)
    off = lax.dot_general(cnt, tri, (((1,), (0,)), ((), ())),
                          preferred_element_type=jnp.float32,
                          precision=lax.Precision.HIGHEST)
    rows = jnp.concatenate(
        [cnt, off, jnp.zeros((6, G), jnp.float32)], axis=0)
    o_ref[...] = rows.astype(jnp.int32)


def _tc_counts(batch):
    b3 = batch.reshape(10, 1, N // 10)
    return pl.pallas_call(
        _tc_counts_body,
        in_specs=[pl.BlockSpec((10, 1, N // 10), lambda: (0, 0, 0))],
        out_specs=pl.BlockSpec((8, G), lambda: (0, 0)),
        out_shape=jax.ShapeDtypeStruct((8, G), jnp.int32),
    )(b3)


def _sc_sortpool_body(v_hbm, h_hbm, co_hbm, out_hbm,
                      vbuf, cvec, ovec, isel, rows, sem):
    c = lax.axis_index("c")
    s = lax.axis_index("s")
    wid = c * NS + s
    g0 = wid * GPT
    pltpu.sync_copy(v_hbm, vbuf)
    pltpu.sync_copy(co_hbm.at[0, pl.ds(g0, GPT)], cvec)
    pltpu.sync_copy(co_hbm.at[1, pl.ds(g0, GPT)], ovec)
    iot = lax.iota(jnp.int32, 16)
    lane0 = iot == 0
    negone = jnp.full((16,), -1.0, jnp.float32)
    big = jnp.full((16,), 1 << 30, jnp.int32)
    cvals = cvec[...]
    ovals = ovec[...]
    for b8 in range(CH // 16):
        isel[pl.ds(b8 * 16, 16)] = jnp.zeros((16,), jnp.int32)
    for gi in range(GPT):
        cnt = cvals[gi]
        off = ovals[gi]
        c0 = off // 16
        c1 = (off + cnt + 15) // 16
        offv = jnp.full((16,), off, jnp.int32)
        cntv = jnp.full((16,), cnt, jnp.int32)
        endv = offv + cntv

        def round_fn(r, _):
            def p1(j, m):
                vv = vbuf[pl.ds(j * 16, 16)]
                pos = jnp.full((16,), j * 16, jnp.int32) + iot
                ok = (pos >= offv) & (pos < endv)
                return jnp.maximum(m, jnp.where(ok, vv, -1.0))

            m = lax.fori_loop(c0, c1, p1, negone)
            mvalv = jnp.full((16,), jnp.max(m), jnp.float32)

            def p2(j, best):
                vv = vbuf[pl.ds(j * 16, 16)]
                pos = jnp.full((16,), j * 16, jnp.int32) + iot
                hit = (pos >= offv) & (pos < endv) & (vv == mvalv)
                return jnp.minimum(best, jnp.where(hit, pos, big))

            best = lax.fori_loop(c0, c1, p2, big)
            bidx = jnp.min(best)
            msk = lane0 & (mvalv >= 0.0)
            bidxv = jnp.full((16,), bidx, jnp.int32)
            plsc.store_scatter(vbuf, [bidxv], negone, mask=msk)
            plsc.store_scatter(isel,
                               [jnp.full((16,), r, jnp.int32) + (gi * K)],
                               bidxv, mask=msk)
            return 0

        lax.fori_loop(0, K, round_fn, 0)
    pltpu.async_copy(h_hbm.at[isel], rows, sem).wait()
    for gi in range(GPT):
        cntv = jnp.full((16,), cvals[gi], jnp.int32)
        for r in range(K):
            w = jnp.where(jnp.full((16,), r, jnp.int32) < cntv, 1.0, 0.0)
            t = gi * K + r
            for cb in range(F // 16):
                rows[t, pl.ds(cb * 16, 16)] = rows[t, pl.ds(cb * 16, 16)] * w
    pltpu.sync_copy(rows.at[pl.ds(0, GPT * K)],
                    out_hbm.at[pl.ds(g0 * K, GPT * K)])


def _sc_sortpool(vlast, h, co):
    fn = pl.kernel(
        _sc_sortpool_body,
        out_type=jax.ShapeDtypeStruct((G * K, F), jnp.float32),
        mesh=_mesh(),
        scratch_types=[
            pltpu.VMEM((N,), jnp.float32),
            pltpu.VMEM((GPT,), jnp.int32),
            pltpu.VMEM((GPT,), jnp.int32),
            pltpu.VMEM((CH,), jnp.int32),
            pltpu.VMEM((CH, F), jnp.float32),
            pltpu.SemaphoreType.DMA,
        ],
        compiler_params=pltpu.CompilerParams(needs_layout_passes=False),
    )
    return fn(vlast, h, co)


def _tc_head_body(d_ref, m_ref, bc_ref, w1_ref, b1_ref, w2_ref, b2_ref,
                  o_ref):
    hp = None
    d = d_ref[...]
    z = lax.dot_general(d, m_ref[...], (((1,), (1,)), ((), ())),
                        preferred_element_type=jnp.float32, precision=hp)
    z = jnp.maximum(z + bc_ref[...], 0.0)
    h1 = lax.dot_general(z, w1_ref[...], (((1,), (1,)), ((), ())),
                         preferred_element_type=jnp.float32, precision=hp)
    h1 = jnp.maximum(h1 + b1_ref[...], 0.0)
    h2 = lax.dot_general(h1, w2_ref[...], (((1,), (1,)), ((), ())),
                         preferred_element_type=jnp.float32, precision=hp)
    h2 = h2 + b2_ref[...]
    mx = jnp.max(h2, axis=1, keepdims=True)
    e = jnp.exp(h2 - mx)
    lse = jnp.log(jnp.sum(e, axis=1, keepdims=True)) + mx
    o_ref[...] = h2 - lse


def _tc_head(d, M, bc, w1, b1, w2, b2):
    co_len = 32 * (K - 2)
    return pl.pallas_call(
        _tc_head_body,
        in_specs=[
            pl.BlockSpec((G, K * F), lambda: (0, 0)),
            pl.BlockSpec((co_len, K * F), lambda: (0, 0)),
            pl.BlockSpec((1, co_len), lambda: (0, 0)),
            pl.BlockSpec((F, co_len), lambda: (0, 0)),
            pl.BlockSpec((1, F), lambda: (0, 0)),
            pl.BlockSpec((10, F), lambda: (0, 0)),
            pl.BlockSpec((1, 10), lambda: (0, 0)),
        ],
        out_specs=pl.BlockSpec((G, 10), lambda: (0, 0)),
        out_shape=jax.ShapeDtypeStruct((G, 10), jnp.float32),
    )(d, M, bc, w1, b1, w2, b2)


def kernel(x, edge_index, batch, sage0_wl, sage0_bl, sage0_wr, sage1_wl,
           sage1_bl, sage1_wr, sage2_wl, sage2_bl, sage2_wr, conv1d_w,
           conv1d_b, lin1_w, lin1_b, lin2_w, lin2_b):
    pad = EPAD - E
    srcp = jnp.concatenate([edge_index[0],
                            jnp.zeros((pad,), jnp.int32)])
    dstp = jnp.concatenate([edge_index[1],
                            DUMP + jnp.arange(pad, dtype=jnp.int32)
                            % (NROW - N)])

    src2 = srcp.reshape(EPAD // CH, CH)
    dst2 = dstp.reshape(EPAD // CH, CH)

    CNT = _sc_count(dstp)
    P = _sc_edge_agg(x, src2, dst2)
    h = _tc_sage(P, CNT, x, sage0_wl, sage0_bl, sage0_wr)
    P = _sc_edge_agg(h, src2, dst2)
    h = _tc_sage(P, CNT, h, sage1_wl, sage1_bl, sage1_wr)
    P = _sc_edge_agg(h, src2, dst2)
    h = _tc_sage(P, CNT, h, sage2_wl, sage2_bl, sage2_wr)

    co = _tc_counts(batch)
    vlast = h[:, F - 1]
    dense = _sc_sortpool(vlast, h, co)

    W = conv1d_w.transpose(0, 2, 1)
    M4 = jnp.zeros((32, K - 2, K, F), jnp.float32)
    for t in range(K - 2):
        M4 = M4.at[:, t, t:t + 3, :].set(W)
    M = M4.reshape(32 * (K - 2), K * F)
    bc = jnp.repeat(conv1d_b, K - 2).reshape(1, 32 * (K - 2))

    return _tc_head(dense.reshape(G, K * F), M, bc, lin1_w,
                    lin1_b.reshape(1, F), lin2_w, lin2_b.reshape(1, 10))


# speedup vs baseline: 1.3518x; 1.3518x over previous
import functools

import jax
import jax.numpy as jnp
from jax import lax
from jax.experimental import pallas as pl
from jax.experimental.pallas import tpu as pltpu
from jax.experimental.pallas import tpu_sc as plsc

N = 10000
F = 128
E = 320000
G = 512
K = 7
NC, NS = 2, 16
NW = NC * NS
CH = 128
EPT = 10240
EPAD = NW * EPT
NROW = 10240
RPT = NROW // NS
DUMP = N
GPT = G // NW
EPT0 = 19456
EPT1 = 1024
NCH0 = EPT0 // CH
NCH1 = EPT1 // CH
NB = 2
GI = 8
CW = 128


def _mesh():
    return plsc.VectorSubcoreMesh(
        core_axis_name="c", subcore_axis_name="s", num_cores=NC,
        num_subcores=NS)


def _sc_edge_body(h_hbm, src2_hbm, dst2_hbm, p_hbm, idx_s2, idx_d2, rows,
                  acc, sem0, sem1):
    sems = (sem0, sem1)
    c = lax.axis_index("c")
    s = lax.axis_index("s")
    row0 = s * RPT
    z16 = jnp.zeros((16,), jnp.float32)

    def zrow(i, _):
        for k8 in range(F // 16):
            rows[0, i, pl.ds(k8 * 16, 16)] = z16
        return 0

    lax.fori_loop(0, CH, zrow, 0)
    for b in range(RPT // CH):
        pltpu.sync_copy(rows.at[0], acc.at[pl.ds(row0 + b * CH, CH)])

    plsc.subcore_barrier()
    nch = jnp.where(c == 0, NCH0, NCH1)
    cbase = jnp.where(c == 0, s * NCH0, NS * NCH0 + s * NCH1)

    def group(g, _):
        cb = cbase + g * GI
        pltpu.sync_copy(src2_hbm.at[pl.ds(cb, GI)], idx_s2)
        pltpu.sync_copy(dst2_hbm.at[pl.ds(cb, GI)], idx_d2)
        for b in range(NB):
            pltpu.async_copy(h_hbm.at[idx_s2.at[b]], rows.at[b], sems[b])

        def inner(j2, _):
            for b in range(NB):
                j = j2 * NB + b
                pltpu.make_async_copy(h_hbm.at[idx_s2.at[j]], rows.at[b],
                                      sems[b]).wait()
                pltpu.sync_copy(rows.at[b], acc.at[idx_d2.at[j]], add=True)
                jn = j + NB

                @pl.when(jn < GI)
                def _():
                    pltpu.async_copy(h_hbm.at[idx_s2.at[jn]], rows.at[b],
                                     sems[b])
            return 0

        lax.fori_loop(0, GI // NB, inner, 0)
        return 0

    lax.fori_loop(0, nch // GI, group, 0)
    plsc.subcore_barrier()
    pltpu.sync_copy(acc.at[pl.ds(row0, RPT)], p_hbm.at[c, pl.ds(row0, RPT)])


def _sc_count_body(dst_hbm, cnt_hbm, idx_d, obuf, cacc):
    c = lax.axis_index("c")
    s = lax.axis_index("s")
    wid = c * NS + s
    row0 = s * RPT
    z16 = jnp.zeros((16,), jnp.float32)

    def zo(i, _):
        for k8 in range(CW // 16):
            obuf[i, pl.ds(k8 * 16, 16)] = z16
        return 0

    lax.fori_loop(0, CH, zo, 0)
    for b in range(RPT // CH):
        pltpu.sync_copy(obuf, cacc.at[pl.ds(row0 + b * CH, CH)])

    def so(i, _):
        for k8 in range(CW // 16):
            obuf[i, pl.ds(k8 * 16, 16)] = jnp.ones((16,), jnp.float32)
        return 0

    lax.fori_loop(0, CH, so, 0)
    plsc.subcore_barrier()
    ebase = wid * EPT

    def chunk(i, _):
        eb = ebase + i * CH
        pltpu.sync_copy(dst_hbm.at[pl.ds(eb, CH)], idx_d)
        pltpu.sync_copy(obuf, cacc.at[idx_d], add=True)
        return 0

    lax.fori_loop(0, EPT // CH, chunk, 0)
    plsc.subcore_barrier()
    pltpu.sync_copy(cacc.at[pl.ds(row0, RPT)],
                    cnt_hbm.at[c, pl.ds(row0, RPT)])


def _sc_count(dstp):
    fn = pl.kernel(
        _sc_count_body,
        out_type=jax.ShapeDtypeStruct((NC, NROW, CW), jnp.float32),
        mesh=_mesh(),
        scratch_types=[
            pltpu.VMEM((CH,), jnp.int32),
            pltpu.VMEM((CH, CW), jnp.float32),
            pltpu.VMEM_SHARED((NROW, CW), jnp.float32),
        ],
        compiler_params=pltpu.CompilerParams(needs_layout_passes=False),
    )
    return fn(dstp)


def _sc_edge_agg(h, src2, dst2):
    fn = pl.kernel(
        _sc_edge_body,
        out_type=jax.ShapeDtypeStruct((NC, NROW, F), jnp.float32),
        mesh=_mesh(),
        scratch_types=[
            pltpu.VMEM((GI, CH), jnp.int32),
            pltpu.VMEM((GI, CH), jnp.int32),
            pltpu.VMEM((NB, CH, F), jnp.float32),
            pltpu.VMEM_SHARED((NROW, F), jnp.float32),
            pltpu.SemaphoreType.DMA,
            pltpu.SemaphoreType.DMA,
        ],
        compiler_params=pltpu.CompilerParams(needs_layout_passes=False),
    )
    return fn(h, src2, dst2)


def _tc_sage_body(p_ref, c_ref, x_ref, wl_ref, wr_ref, bl_ref, o_ref):
    p = p_ref[0] + p_ref[1]
    cc = c_ref[0, :, 0:1] + c_ref[1, :, 0:1]
    agg = p / jnp.maximum(cc, 1.0)
    acc = lax.dot_general(agg, wl_ref[...], (((1,), (1,)), ((), ())),
                          preferred_element_type=jnp.float32)
    acc = acc + lax.dot_general(x_ref[...], wr_ref[...],
                                (((1,), (1,)), ((), ())),
                                preferred_element_type=jnp.float32)
    o_ref[...] = jnp.maximum(acc + bl_ref[...], 0.0)


def _tc_sage(P, CNT, x, wl, bl, wr):
    nb = 10
    br = N // nb
    return pl.pallas_call(
        _tc_sage_body,
        grid=(nb,),
        in_specs=[
            pl.BlockSpec((NC, br, F), lambda i: (0, i, 0)),
            pl.BlockSpec((NC, br, CW), lambda i: (0, i, 0)),
            pl.BlockSpec((br, F), lambda i: (i, 0)),
            pl.BlockSpec((F, F), lambda i: (0, 0)),
            pl.BlockSpec((F, F), lambda i: (0, 0)),
            pl.BlockSpec((1, F), lambda i: (0, 0)),
        ],
        out_specs=pl.BlockSpec((br, F), lambda i: (i, 0)),
        out_shape=jax.ShapeDtypeStruct((N, F), jnp.float32),
    )(P, CNT, x, wl, wr, bl.reshape(1, F))


def _tc_counts_body(b_ref, o_ref):
    gids = lax.broadcasted_iota(jnp.int32, (1, G), 1)
    cnt = jnp.zeros((1, G), jnp.float32)
    for i in range(10):
        bi = b_ref[i]
        eq = (bi[0][:, None] == gids).astype(jnp.float32)
        cnt = cnt + jnp.sum(eq, axis=0, keepdims=True)
    rl = lax.broadcasted_iota(jnp.int32, (G, G), 0)
    cl = lax.broadcasted_iota(jnp.int32, (G, G), 1)
    tri = (rl < cl).astype(jnp.float32)
    off = lax.dot_general(cnt, tri, (((1,), (0,)), ((), ())),
                          preferred_element_type=jnp.float32,
                          precision=lax.Precision.HIGHEST)
    rows = jnp.concatenate(
        [cnt, off, jnp.zeros((6, G), jnp.float32)], axis=0)
    o_ref[...] = rows.astype(jnp.int32)


def _tc_counts(batch):
    b3 = batch.reshape(10, 1, N // 10)
    return pl.pallas_call(
        _tc_counts_body,
        in_specs=[pl.BlockSpec((10, 1, N // 10), lambda: (0, 0, 0))],
        out_specs=pl.BlockSpec((8, G), lambda: (0, 0)),
        out_shape=jax.ShapeDtypeStruct((8, G), jnp.int32),
    )(b3)


def _sc_sortpool_body(v_hbm, h_hbm, co_hbm, out_hbm,
                      vbuf, cvec, ovec, isel, rows, sem):
    c = lax.axis_index("c")
    s = lax.axis_index("s")
    wid = c * NS + s
    g0 = wid * GPT
    pltpu.sync_copy(v_hbm, vbuf)
    pltpu.sync_copy(co_hbm.at[0, pl.ds(g0, GPT)], cvec)
    pltpu.sync_copy(co_hbm.at[1, pl.ds(g0, GPT)], ovec)
    iot = lax.iota(jnp.int32, 16)
    lane0 = iot == 0
    negone = jnp.full((16,), -1.0, jnp.float32)
    big = jnp.full((16,), 1 << 30, jnp.int32)
    cvals = cvec[...]
    ovals = ovec[...]
    for b8 in range(CH // 16):
        isel[pl.ds(b8 * 16, 16)] = jnp.zeros((16,), jnp.int32)
    for gi in range(GPT):
        cnt = cvals[gi]
        off = ovals[gi]
        c0 = off // 16
        c1 = (off + cnt + 15) // 16
        offv = jnp.full((16,), off, jnp.int32)
        cntv = jnp.full((16,), cnt, jnp.int32)
        endv = offv + cntv

        def round_fn(r, _):
            def p1(j, m):
                vv = vbuf[pl.ds(j * 16, 16)]
                pos = jnp.full((16,), j * 16, jnp.int32) + iot
                ok = (pos >= offv) & (pos < endv)
                return jnp.maximum(m, jnp.where(ok, vv, -1.0))

            m = lax.fori_loop(c0, c1, p1, negone)
            mvalv = jnp.full((16,), jnp.max(m), jnp.float32)

            def p2(j, best):
                vv = vbuf[pl.ds(j * 16, 16)]
                pos = jnp.full((16,), j * 16, jnp.int32) + iot
                hit = (pos >= offv) & (pos < endv) & (vv == mvalv)
                return jnp.minimum(best, jnp.where(hit, pos, big))

            best = lax.fori_loop(c0, c1, p2, big)
            bidx = jnp.min(best)
            msk = lane0 & (mvalv >= 0.0)
            bidxv = jnp.full((16,), bidx, jnp.int32)
            plsc.store_scatter(vbuf, [bidxv], negone, mask=msk)
            plsc.store_scatter(isel,
                               [jnp.full((16,), r, jnp.int32) + (gi * K)],
                               bidxv, mask=msk)
            return 0

        lax.fori_loop(0, K, round_fn, 0)
    pltpu.async_copy(h_hbm.at[isel], rows, sem).wait()
    for gi in range(GPT):
        cntv = jnp.full((16,), cvals[gi], jnp.int32)
        for r in range(K):
            w = jnp.where(jnp.full((16,), r, jnp.int32) < cntv, 1.0, 0.0)
            t = gi * K + r
            for cb in range(F // 16):
                rows[t, pl.ds(cb * 16, 16)] = rows[t, pl.ds(cb * 16, 16)] * w
    pltpu.sync_copy(rows.at[pl.ds(0, GPT * K)],
                    out_hbm.at[pl.ds(g0 * K, GPT * K)])


def _sc_sortpool(vlast, h, co):
    fn = pl.kernel(
        _sc_sortpool_body,
        out_type=jax.ShapeDtypeStruct((G * K, F), jnp.float32),
        mesh=_mesh(),
        scratch_types=[
            pltpu.VMEM((N,), jnp.float32),
            pltpu.VMEM((GPT,), jnp.int32),
            pltpu.VMEM((GPT,), jnp.int32),
            pltpu.VMEM((CH,), jnp.int32),
            pltpu.VMEM((CH, F), jnp.float32),
            pltpu.SemaphoreType.DMA,
        ],
        compiler_params=pltpu.CompilerParams(needs_layout_passes=False),
    )
    return fn(vlast, h, co)


def _tc_head_body(d_ref, m_ref, bc_ref, w1_ref, b1_ref, w2_ref, b2_ref,
                  o_ref):
    hp = None
    d = d_ref[...]
    z = lax.dot_general(d, m_ref[...], (((1,), (1,)), ((), ())),
                        preferred_element_type=jnp.float32, precision=hp)
    z = jnp.maximum(z + bc_ref[...], 0.0)
    h1 = lax.dot_general(z, w1_ref[...], (((1,), (1,)), ((), ())),
                         preferred_element_type=jnp.float32, precision=hp)
    h1 = jnp.maximum(h1 + b1_ref[...], 0.0)
    h2 = lax.dot_general(h1, w2_ref[...], (((1,), (1,)), ((), ())),
                         preferred_element_type=jnp.float32, precision=hp)
    h2 = h2 + b2_ref[...]
    mx = jnp.max(h2, axis=1, keepdims=True)
    e = jnp.exp(h2 - mx)
    lse = jnp.log(jnp.sum(e, axis=1, keepdims=True)) + mx
    o_ref[...] = h2 - lse


def _tc_head(d, M, bc, w1, b1, w2, b2):
    co_len = 32 * (K - 2)
    return pl.pallas_call(
        _tc_head_body,
        in_specs=[
            pl.BlockSpec((G, K * F), lambda: (0, 0)),
            pl.BlockSpec((co_len, K * F), lambda: (0, 0)),
            pl.BlockSpec((1, co_len), lambda: (0, 0)),
            pl.BlockSpec((F, co_len), lambda: (0, 0)),
            pl.BlockSpec((1, F), lambda: (0, 0)),
            pl.BlockSpec((10, F), lambda: (0, 0)),
            pl.BlockSpec((1, 10), lambda: (0, 0)),
        ],
        out_specs=pl.BlockSpec((G, 10), lambda: (0, 0)),
        out_shape=jax.ShapeDtypeStruct((G, 10), jnp.float32),
    )(d, M, bc, w1, b1, w2, b2)


def kernel(x, edge_index, batch, sage0_wl, sage0_bl, sage0_wr, sage1_wl,
           sage1_bl, sage1_wr, sage2_wl, sage2_bl, sage2_wr, conv1d_w,
           conv1d_b, lin1_w, lin1_b, lin2_w, lin2_b):
    pad = EPAD - E
    srcp = jnp.concatenate([edge_index[0],
                            jnp.zeros((pad,), jnp.int32)])
    dstp = jnp.concatenate([edge_index[1],
                            DUMP + jnp.arange(pad, dtype=jnp.int32)
                            % (NROW - N)])

    src2 = srcp.reshape(EPAD // CH, CH)
    dst2 = dstp.reshape(EPAD // CH, CH)

    CNT = _sc_count(dstp)
    P = _sc_edge_agg(x, src2, dst2)
    h = _tc_sage(P, CNT, x, sage0_wl, sage0_bl, sage0_wr)
    P = _sc_edge_agg(h, src2, dst2)
    h = _tc_sage(P, CNT, h, sage1_wl, sage1_bl, sage1_wr)
    P = _sc_edge_agg(h, src2, dst2)
    h = _tc_sage(P, CNT, h, sage2_wl, sage2_bl, sage2_wr)

    co = _tc_counts(batch)
    vlast = h[:, F - 1]
    dense = _sc_sortpool(vlast, h, co)

    W = conv1d_w.transpose(0, 2, 1)
    M4 = jnp.zeros((32, K - 2, K, F), jnp.float32)
    for t in range(K - 2):
        M4 = M4.at[:, t, t:t + 3, :].set(W)
    M = M4.reshape(32 * (K - 2), K * F)
    bc = jnp.repeat(conv1d_b, K - 2).reshape(1, 32 * (K - 2))

    return _tc_head(dense.reshape(G, K * F), M, bc, lin1_w,
                    lin1_b.reshape(1, F), lin2_w, lin2_b.reshape(1, 10))
